# baseline (speedup 1.0000x reference)
import functools

import jax
import jax.numpy as jnp
from jax import lax
from jax.experimental import pallas as pl
from jax.experimental.pallas import tpu as pltpu
from jax.experimental.pallas import tpu_sc as plsc

_EPS = 1e-5
_NC = 2
_NS = 16
_NW = _NC * _NS


def _sc_gather(emb2_flat, emb1_flat, flat_idx):
    FV, D = emb2_flat.shape
    N = flat_idx.shape[0]
    assert N % (_NW * 8) == 0
    PW = N // _NW
    CH = PW
    while CH * D * 4 > 131072 or CH % 8 != 0:
        for c in range(CH - 1, 0, -1):
            if PW % c == 0:
                CH = c
                break
    n_chunks = PW // CH

    @functools.partial(
        pl.kernel,
        mesh=plsc.VectorSubcoreMesh(core_axis_name="c", subcore_axis_name="s"),
        compiler_params=pltpu.CompilerParams(use_tc_tiling_on_sc=False),
        out_type=[
            jax.ShapeDtypeStruct((N, D), jnp.float32),
            jax.ShapeDtypeStruct((N,), jnp.float32),
        ],
        scratch_types=[
            pltpu.VMEM((PW,), jnp.int32),
            pltpu.VMEM((CH, D), jnp.float32),
            pltpu.VMEM((CH,), jnp.float32),
            pltpu.SemaphoreType.DMA,
            pltpu.SemaphoreType.DMA,
        ],
    )
    def k(emb2_h, emb1_h, idx_h, rows_o, fo_o, idx_v, rows_v, fo_v, sem_r, sem_s):
        wid = lax.axis_index("s") * _NC + lax.axis_index("c")
        base = wid * PW
        pltpu.sync_copy(idx_h.at[pl.ds(base, PW)], idx_v)
        for j in range(n_chunks):
            off = j * CH
            iv = idx_v.at[pl.ds(off, CH)]
            pltpu.async_copy(emb2_h.at[iv], rows_v, sem_r).wait()
            pltpu.sync_copy(rows_v, rows_o.at[pl.ds(base + off, CH)])
            pltpu.async_copy(emb1_h.at[iv], fo_v, sem_s).wait()
            pltpu.sync_copy(fo_v, fo_o.at[pl.ds(base + off, CH)])

    return k(emb2_flat, emb1_flat, flat_idx)


def _stage_first(emb, fo26, dense, S, W0e, W0d, b0, WdT, cbias, bb):
    Bn, E = emb.shape
    H = W0e.shape[1]
    nb = Bn // bb

    def body(emb_r, fo_r, dn_r, S_r, W0e_r, W0d_r, b0_r, WdT_r, cb_r,
             y0_r, st_r, fofm_r):
        i = pl.program_id(0)
        x = emb_r[...]
        dn = dn_r[...]
        y = jnp.dot(x, W0e_r[...], preferred_element_type=jnp.float32)
        y = y + jnp.dot(dn, W0d_r[...], preferred_element_type=jnp.float32)
        y = y + b0_r[...]
        y0_r[...] = y
        se = jnp.dot(x, S_r[...], preferred_element_type=jnp.float32)
        sq = jnp.dot(x * x, S_r[...], preferred_element_type=jnp.float32)
        fm = 0.5 * jnp.sum(se * se - sq, axis=1, keepdims=True)
        fo = jnp.sum(fo_r[...], axis=1, keepdims=True)
        fo = fo + jnp.dot(dn, WdT_r[...], preferred_element_type=jnp.float32)
        fofm_r[...] = fm + fo + cb_r[...]
        st = jnp.concatenate(
            [jnp.sum(y, axis=0, keepdims=True),
             jnp.sum(y * y, axis=0, keepdims=True)], axis=0)

        @pl.when(i == 0)
        def _():
            st_r[...] = st

        @pl.when(i != 0)
        def _():
            st_r[...] = st_r[...] + st

    return pl.pallas_call(
        body,
        grid=(nb,),
        in_specs=[
            pl.BlockSpec((bb, E), lambda i: (i, 0)),
            pl.BlockSpec((bb, fo26.shape[1]), lambda i: (i, 0)),
            pl.BlockSpec((bb, dense.shape[1]), lambda i: (i, 0)),
            pl.BlockSpec(S.shape, lambda i: (0, 0)),
            pl.BlockSpec(W0e.shape, lambda i: (0, 0)),
            pl.BlockSpec(W0d.shape, lambda i: (0, 0)),
            pl.BlockSpec(b0.shape, lambda i: (0, 0)),
            pl.BlockSpec(WdT.shape, lambda i: (0, 0)),
            pl.BlockSpec(cbias.shape, lambda i: (0, 0)),
        ],
        out_specs=[
            pl.BlockSpec((bb, H), lambda i: (i, 0)),
            pl.BlockSpec((2, H), lambda i: (0, 0)),
            pl.BlockSpec((bb, 1), lambda i: (i, 0)),
        ],
        out_shape=[
            jax.ShapeDtypeStruct((Bn, H), jnp.float32),
            jax.ShapeDtypeStruct((2, H), jnp.float32),
            jax.ShapeDtypeStruct((Bn, 1), jnp.float32),
        ],
    )(emb, fo26, dense, S, W0e, W0d, b0, WdT, cbias)


def _stage_hidden(yp, stats, g, beta, WT, b, bb):
    Bn, H = yp.shape
    H1 = WT.shape[1]
    nb = Bn // bb
    inv_b = 1.0 / Bn

    def body(yp_r, st_r, g_r, be_r, W_r, b_r, y_r, sto_r):
        i = pl.program_id(0)
        mean = st_r[0:1, :] * inv_b
        var = st_r[1:2, :] * inv_b - mean * mean
        scale = g_r[...] * lax.rsqrt(var + _EPS)
        shift = be_r[...] - mean * scale
        h = jnp.maximum(yp_r[...] * scale + shift, 0.0)
        y = jnp.dot(h, W_r[...], preferred_element_type=jnp.float32) + b_r[...]
        y_r[...] = y
        st = jnp.concatenate(
            [jnp.sum(y, axis=0, keepdims=True),
             jnp.sum(y * y, axis=0, keepdims=True)], axis=0)

        @pl.when(i == 0)
        def _():
            sto_r[...] = st

        @pl.when(i != 0)
        def _():
            sto_r[...] = sto_r[...] + st

    return pl.pallas_call(
        body,
        grid=(nb,),
        in_specs=[
            pl.BlockSpec((bb, H), lambda i: (i, 0)),
            pl.BlockSpec((2, H), lambda i: (0, 0)),
            pl.BlockSpec(g.shape, lambda i: (0, 0)),
            pl.BlockSpec(beta.shape, lambda i: (0, 0)),
            pl.BlockSpec(WT.shape, lambda i: (0, 0)),
            pl.BlockSpec(b.shape, lambda i: (0, 0)),
        ],
        out_specs=[
            pl.BlockSpec((bb, H1), lambda i: (i, 0)),
            pl.BlockSpec((2, H1), lambda i: (0, 0)),
        ],
        out_shape=[
            jax.ShapeDtypeStruct((Bn, H1), jnp.float32),
            jax.ShapeDtypeStruct((2, H1), jnp.float32),
        ],
    )(yp, stats, g, beta, WT, b)


def _stage_final(yp, stats, g, beta, WoT, fofm, cbias, bb):
    Bn, H = yp.shape
    nb = Bn // bb
    inv_b = 1.0 / Bn

    def body(yp_r, st_r, g_r, be_r, Wo_r, ff_r, cb_r, out_r):
        mean = st_r[0:1, :] * inv_b
        var = st_r[1:2, :] * inv_b - mean * mean
        scale = g_r[...] * lax.rsqrt(var + _EPS)
        shift = be_r[...] - mean * scale
        h = jnp.maximum(yp_r[...] * scale + shift, 0.0)
        z = jnp.dot(h, Wo_r[...], preferred_element_type=jnp.float32)
        z = z + ff_r[...] + cb_r[...]
        out_r[...] = 1.0 / (1.0 + jnp.exp(-z))

    return pl.pallas_call(
        body,
        grid=(nb,),
        in_specs=[
            pl.BlockSpec((bb, H), lambda i: (i, 0)),
            pl.BlockSpec((2, H), lambda i: (0, 0)),
            pl.BlockSpec(g.shape, lambda i: (0, 0)),
            pl.BlockSpec(beta.shape, lambda i: (0, 0)),
            pl.BlockSpec(WoT.shape, lambda i: (0, 0)),
            pl.BlockSpec((bb, 1), lambda i: (i, 0)),
            pl.BlockSpec(cbias.shape, lambda i: (0, 0)),
        ],
        out_specs=pl.BlockSpec((bb, 1), lambda i: (i, 0)),
        out_shape=jax.ShapeDtypeStruct((Bn, 1), jnp.float32),
    )(yp, stats, g, beta, WoT, fofm, cbias)


def kernel(sparse_inputs, dense_inputs, emb1, emb2, Wd, bd,
           W0, b0, g0, beta0, W1, b1, g1, beta1, W2, b2, g2, beta2,
           Wo, bo, bias0):
    Bn, F = sparse_inputs.shape
    _, V, D = emb2.shape
    E = F * D

    flat_idx = (sparse_inputs.astype(jnp.int32)
                + (jnp.arange(F, dtype=jnp.int32) * V)[None, :]).reshape(-1)
    rows, fo_sc = _sc_gather(emb2.reshape(F * V, D), emb1.reshape(F * V),
                             flat_idx)
    emb_flat = rows.reshape(Bn, E)
    fo26 = fo_sc.reshape(Bn, F)

    S = jnp.tile(jnp.eye(D, dtype=jnp.float32), (F, 1))
    W0T = W0.T
    W0e, W0d = W0T[:E], W0T[E:]
    cbias_a = (bd + bias0).reshape(1, 1)

    bb = 1024
    y0, st0, fofm = _stage_first(emb_flat, fo26, dense_inputs, S,
                                 W0e, W0d, b0.reshape(1, -1), Wd.T,
                                 cbias_a, bb)
    y1, st1 = _stage_hidden(y0, st0, g0.reshape(1, -1), beta0.reshape(1, -1),
                            W1.T, b1.reshape(1, -1), bb)
    y2, st2 = _stage_hidden(y1, st1, g1.reshape(1, -1), beta1.reshape(1, -1),
                            W2.T, b2.reshape(1, -1), bb)
    out = _stage_final(y2, st2, g2.reshape(1, -1), beta2.reshape(1, -1),
                       Wo.T, fofm, bo.reshape(1, 1), bb)
    return out.reshape(Bn)


# speedup vs baseline: 4.9696x; 4.9696x over previous
import functools

import jax
import jax.numpy as jnp
from jax import lax
from jax.experimental import pallas as pl
from jax.experimental.pallas import tpu as pltpu
from jax.experimental.pallas import tpu_sc as plsc

_EPS = 1e-5
_NC = 2
_NS = 16
_NW = _NC * _NS


def _sc_gather_t(emb2_rows, emb1_rows, idx_t):
    R2, V = emb2_rows.shape
    F, B = idx_t.shape
    D = R2 // F
    NJ = R2 // _NW
    QC = 4096
    NQ = B // QC
    GRP = 8

    @functools.partial(
        pl.kernel,
        mesh=plsc.VectorSubcoreMesh(core_axis_name="c", subcore_axis_name="s"),
        compiler_params=pltpu.CompilerParams(use_tc_tiling_on_sc=True,
                                             needs_layout_passes=False),
        out_type=[
            jax.ShapeDtypeStruct((R2, B), jnp.float32),
            jax.ShapeDtypeStruct((F, B), jnp.float32),
        ],
        scratch_types=[
            pltpu.VMEM((V,), jnp.float32),
            pltpu.VMEM((B,), jnp.int32),
            pltpu.VMEM((QC,), jnp.float32),
            pltpu.VMEM((QC,), jnp.float32),
            pltpu.SemaphoreType.DMA,
            pltpu.SemaphoreType.DMA,
            pltpu.SemaphoreType.DMA,
        ],
    )
    def k(e2_h, e1_h, idx_h, out2_h, out1_h,
          row_v, idx_v, ob0, ob1, sem_in, semw0, semw1):
        wid = lax.axis_index("s") * _NC + lax.axis_index("c")
        r0 = wid * NJ
        obufs = (ob0, ob1)
        semws = (semw0, semw1)
        pending = [None, None]

        def gather_quarter(q, ob):
            def body(i, _):
                base = q * QC + i * (GRP * 16)
                obase = i * (GRP * 16)
                for g in range(GRP):
                    iv = idx_v[pl.ds(base + g * 16, 16)]
                    ob[pl.ds(obase + g * 16, 16)] = plsc.load_gather(row_v, [iv])
                return 0
            lax.fori_loop(0, QC // (GRP * 16), body, 0)

        def do_row(out_h, r):
            for q in range(NQ):
                kq = q % 2
                if pending[kq] is not None:
                    pending[kq].wait()
                gather_quarter(q, obufs[kq])
                pending[kq] = pltpu.async_copy(
                    obufs[kq], out_h.at[r, pl.ds(q * QC, QC)], semws[kq])

        for j in range(NJ):
            r = r0 + j
            f = r // D
            if j == 0:
                pltpu.sync_copy(idx_h.at[f], idx_v)
            else:
                fprev = (r0 + j - 1) // D

                @pl.when(f != fprev)
                def _():
                    pltpu.sync_copy(idx_h.at[f], idx_v)

            pltpu.sync_copy(e2_h.at[r], row_v)
            do_row(out2_h, r)

        for kq in range(2):
            if pending[kq] is not None:
                pending[kq].wait()

        @pl.when(wid < F)
        def _():
            pltpu.sync_copy(idx_h.at[wid], idx_v)
            pltpu.sync_copy(e1_h.at[wid], row_v)
            for q in range(NQ):
                gather_quarter(q, obufs[q % 2])
                pltpu.sync_copy(obufs[q % 2],
                                out1_h.at[wid, pl.ds(q * QC, QC)])

    return k(emb2_rows, emb1_rows, idx_t)


def _stage_first_t(emb_t, fo_t, dense_t, S_t, W0e, W0d, b0c, Wdr, cbias, bb):
    E, Bn = emb_t.shape
    H = W0e.shape[0]
    nb = Bn // bb

    def body(emb_r, fo_r, dn_r, S_r, W0e_r, W0d_r, b0_r, Wd_r, cb_r,
             y0_r, ss_r, sq_r, fofm_r):
        i = pl.program_id(0)
        x = emb_r[...]
        dn = dn_r[...]
        y = jnp.dot(W0e_r[...], x, preferred_element_type=jnp.float32)
        y = y + jnp.dot(W0d_r[...], dn, preferred_element_type=jnp.float32)
        y = y + b0_r[...]
        y0_r[...] = y
        se = jnp.dot(S_r[...], x, preferred_element_type=jnp.float32)
        sq = jnp.dot(S_r[...], x * x, preferred_element_type=jnp.float32)
        fm = 0.5 * jnp.sum(se * se - sq, axis=0, keepdims=True)
        fo = jnp.sum(fo_r[...], axis=0, keepdims=True)
        fo = fo + jnp.dot(Wd_r[...], dn, preferred_element_type=jnp.float32)
        fofm_r[...] = fm + fo + cb_r[...]
        s1 = jnp.sum(y, axis=1, keepdims=True)
        s2 = jnp.sum(y * y, axis=1, keepdims=True)

        @pl.when(i == 0)
        def _():
            ss_r[...] = s1
            sq_r[...] = s2

        @pl.when(i != 0)
        def _():
            ss_r[...] = ss_r[...] + s1
            sq_r[...] = sq_r[...] + s2

    return pl.pallas_call(
        body,
        grid=(nb,),
        in_specs=[
            pl.BlockSpec((E, bb), lambda i: (0, i)),
            pl.BlockSpec((fo_t.shape[0], bb), lambda i: (0, i)),
            pl.BlockSpec((dense_t.shape[0], bb), lambda i: (0, i)),
            pl.BlockSpec(S_t.shape, lambda i: (0, 0)),
            pl.BlockSpec(W0e.shape, lambda i: (0, 0)),
            pl.BlockSpec(W0d.shape, lambda i: (0, 0)),
            pl.BlockSpec(b0c.shape, lambda i: (0, 0)),
            pl.BlockSpec(Wdr.shape, lambda i: (0, 0)),
            pl.BlockSpec(cbias.shape, lambda i: (0, 0)),
        ],
        out_specs=[
            pl.BlockSpec((H, bb), lambda i: (0, i)),
            pl.BlockSpec((H, 1), lambda i: (0, 0)),
            pl.BlockSpec((H, 1), lambda i: (0, 0)),
            pl.BlockSpec((1, bb), lambda i: (0, i)),
        ],
        out_shape=[
            jax.ShapeDtypeStruct((H, Bn), jnp.float32),
            jax.ShapeDtypeStruct((H, 1), jnp.float32),
            jax.ShapeDtypeStruct((H, 1), jnp.float32),
            jax.ShapeDtypeStruct((1, Bn), jnp.float32),
        ],
    )(emb_t, fo_t, dense_t, S_t, W0e, W0d, b0c, Wdr, cbias)


def _stage_hidden_t(yp, ss, sq, gc, bec, W, bc, bb):
    H, Bn = yp.shape
    H1 = W.shape[0]
    nb = Bn // bb
    inv_b = 1.0 / Bn

    def body(yp_r, ss_r, sq_r, g_r, be_r, W_r, b_r, y_r, oss_r, osq_r):
        i = pl.program_id(0)
        mean = ss_r[...] * inv_b
        var = sq_r[...] * inv_b - mean * mean
        scale = g_r[...] * lax.rsqrt(var + _EPS)
        shift = be_r[...] - mean * scale
        h = jnp.maximum(yp_r[...] * scale + shift, 0.0)
        y = jnp.dot(W_r[...], h, preferred_element_type=jnp.float32) + b_r[...]
        y_r[...] = y
        s1 = jnp.sum(y, axis=1, keepdims=True)
        s2 = jnp.sum(y * y, axis=1, keepdims=True)

        @pl.when(i == 0)
        def _():
            oss_r[...] = s1
            osq_r[...] = s2

        @pl.when(i != 0)
        def _():
            oss_r[...] = oss_r[...] + s1
            osq_r[...] = osq_r[...] + s2

    return pl.pallas_call(
        body,
        grid=(nb,),
        in_specs=[
            pl.BlockSpec((H, bb), lambda i: (0, i)),
            pl.BlockSpec((H, 1), lambda i: (0, 0)),
            pl.BlockSpec((H, 1), lambda i: (0, 0)),
            pl.BlockSpec(gc.shape, lambda i: (0, 0)),
            pl.BlockSpec(bec.shape, lambda i: (0, 0)),
            pl.BlockSpec(W.shape, lambda i: (0, 0)),
            pl.BlockSpec(bc.shape, lambda i: (0, 0)),
        ],
        out_specs=[
            pl.BlockSpec((H1, bb), lambda i: (0, i)),
            pl.BlockSpec((H1, 1), lambda i: (0, 0)),
            pl.BlockSpec((H1, 1), lambda i: (0, 0)),
        ],
        out_shape=[
            jax.ShapeDtypeStruct((H1, Bn), jnp.float32),
            jax.ShapeDtypeStruct((H1, 1), jnp.float32),
            jax.ShapeDtypeStruct((H1, 1), jnp.float32),
        ],
    )(yp, ss, sq, gc, bec, W, bc)


def _stage_final_t(yp, ss, sq, gc, bec, Wor, fofm, cbias, bb):
    H, Bn = yp.shape
    nb = Bn // bb
    inv_b = 1.0 / Bn

    def body(yp_r, ss_r, sq_r, g_r, be_r, Wo_r, ff_r, cb_r, out_r):
        mean = ss_r[...] * inv_b
        var = sq_r[...] * inv_b - mean * mean
        scale = g_r[...] * lax.rsqrt(var + _EPS)
        shift = be_r[...] - mean * scale
        h = jnp.maximum(yp_r[...] * scale + shift, 0.0)
        z = jnp.dot(Wo_r[...], h, preferred_element_type=jnp.float32)
        z = z + ff_r[...] + cb_r[...]
        out_r[...] = 1.0 / (1.0 + jnp.exp(-z))

    return pl.pallas_call(
        body,
        grid=(nb,),
        in_specs=[
            pl.BlockSpec((H, bb), lambda i: (0, i)),
            pl.BlockSpec((H, 1), lambda i: (0, 0)),
            pl.BlockSpec((H, 1), lambda i: (0, 0)),
            pl.BlockSpec(gc.shape, lambda i: (0, 0)),
            pl.BlockSpec(bec.shape, lambda i: (0, 0)),
            pl.BlockSpec(Wor.shape, lambda i: (0, 0)),
            pl.BlockSpec((1, bb), lambda i: (0, i)),
            pl.BlockSpec(cbias.shape, lambda i: (0, 0)),
        ],
        out_specs=pl.BlockSpec((1, bb), lambda i: (0, i)),
        out_shape=jax.ShapeDtypeStruct((1, Bn), jnp.float32),
    )(yp, ss, sq, gc, bec, Wor, fofm, cbias)


def kernel(sparse_inputs, dense_inputs, emb1, emb2, Wd, bd,
           W0, b0, g0, beta0, W1, b1, g1, beta1, W2, b2, g2, beta2,
           Wo, bo, bias0):
    Bn, F = sparse_inputs.shape
    _, V, D = emb2.shape
    E = F * D

    emb2_rows = emb2.transpose(0, 2, 1).reshape(E, V)
    emb1_rows = emb1.transpose(0, 2, 1).reshape(F, V)
    idx_t = sparse_inputs.T
    dense_t = dense_inputs.T

    emb_t, fo_t = _sc_gather_t(emb2_rows, emb1_rows, idx_t)

    S_t = jnp.tile(jnp.eye(D, dtype=jnp.float32), (1, F))
    W0e, W0d = W0[:, :E], W0[:, E:]
    cbias_a = (bd + bias0).reshape(1, 1)

    bb = 2048
    y0, ss0, sq0, fofm = _stage_first_t(emb_t, fo_t, dense_t, S_t,
                                        W0e, W0d, b0.reshape(-1, 1), Wd,
                                        cbias_a, bb)
    y1, ss1, sq1 = _stage_hidden_t(y0, ss0, sq0, g0.reshape(-1, 1),
                                   beta0.reshape(-1, 1), W1,
                                   b1.reshape(-1, 1), bb)
    y2, ss2, sq2 = _stage_hidden_t(y1, ss1, sq1, g1.reshape(-1, 1),
                                   beta1.reshape(-1, 1), W2,
                                   b2.reshape(-1, 1), bb)
    out = _stage_final_t(y2, ss2, sq2, g2.reshape(-1, 1),
                         beta2.reshape(-1, 1), Wo, fofm,
                         bo.reshape(1, 1), bb)
    return out.reshape(Bn)


# speedup vs baseline: 4.9812x; 1.0023x over previous
import functools

import jax
import jax.numpy as jnp
from jax import lax
from jax.experimental import pallas as pl
from jax.experimental.pallas import tpu as pltpu
from jax.experimental.pallas import tpu_sc as plsc

_EPS = 1e-5
_NC = 2
_NS = 16
_NW = _NC * _NS


def _sc_gather_t(emb2_rows, emb1_rows, idx_t):
    R2, V = emb2_rows.shape
    F, B = idx_t.shape
    D = R2 // F
    NJ = R2 // _NW
    QC = 4096
    NQ = B // QC
    GRP = 8

    @functools.partial(
        pl.kernel,
        mesh=plsc.VectorSubcoreMesh(core_axis_name="c", subcore_axis_name="s"),
        compiler_params=pltpu.CompilerParams(use_tc_tiling_on_sc=True,
                                             needs_layout_passes=False),
        out_type=[
            jax.ShapeDtypeStruct((R2, B), jnp.float32),
            jax.ShapeDtypeStruct((F, B), jnp.float32),
        ],
        scratch_types=[
            pltpu.VMEM((V,), jnp.float32),
            pltpu.VMEM((B,), jnp.int32),
            pltpu.VMEM((QC,), jnp.float32),
            pltpu.VMEM((QC,), jnp.float32),
            pltpu.SemaphoreType.DMA,
            pltpu.SemaphoreType.DMA,
            pltpu.SemaphoreType.DMA,
        ],
    )
    def k(e2_h, e1_h, idx_h, out2_h, out1_h,
          row_v, idx_v, ob0, ob1, sem_in, semw0, semw1):
        wid = lax.axis_index("s") * _NC + lax.axis_index("c")
        r0 = wid * NJ
        obufs = (ob0, ob1)
        semws = (semw0, semw1)
        pending = [None, None]

        def gather_quarter(q, ob):
            def body(i, _):
                base = q * QC + i * (GRP * 16)
                obase = i * (GRP * 16)
                for g in range(GRP):
                    iv = idx_v[pl.ds(base + g * 16, 16)]
                    ob[pl.ds(obase + g * 16, 16)] = plsc.load_gather(row_v, [iv])
                return 0
            lax.fori_loop(0, QC // (GRP * 16), body, 0)

        def do_row(out_h, r):
            for q in range(NQ):
                kq = q % 2
                if pending[kq] is not None:
                    pending[kq].wait()
                gather_quarter(q, obufs[kq])
                pending[kq] = pltpu.async_copy(
                    obufs[kq], out_h.at[r, pl.ds(q * QC, QC)], semws[kq])

        for j in range(NJ):
            r = r0 + j
            f = r // D
            if j == 0:
                pltpu.sync_copy(idx_h.at[f], idx_v)
            else:
                fprev = (r0 + j - 1) // D

                @pl.when(f != fprev)
                def _():
                    pltpu.sync_copy(idx_h.at[f], idx_v)

            pltpu.sync_copy(e2_h.at[r], row_v)
            do_row(out2_h, r)

        for kq in range(2):
            if pending[kq] is not None:
                pending[kq].wait()

        @pl.when(wid < F)
        def _():
            pltpu.sync_copy(idx_h.at[wid], idx_v)
            pltpu.sync_copy(e1_h.at[wid], row_v)
            for q in range(NQ):
                gather_quarter(q, obufs[q % 2])
                pltpu.sync_copy(obufs[q % 2],
                                out1_h.at[wid, pl.ds(q * QC, QC)])

    return k(emb2_rows, emb1_rows, idx_t)


def _stage_first_t(emb_t, fo_t, dense_t, S_t, W0e, W0d, b0c, Wdr, cbias, bb):
    E, Bn = emb_t.shape
    H = W0e.shape[0]
    nb = Bn // bb

    def body(emb_r, fo_r, dn_r, S_r, W0e_r, W0d_r, b0_r, Wd_r, cb_r,
             y0_r, ss_r, sq_r, fofm_r):
        i = pl.program_id(0)
        x = emb_r[...]
        dn = dn_r[...]
        xb = x.astype(jnp.bfloat16)
        y = jnp.dot(W0e_r[...], xb, preferred_element_type=jnp.float32)
        y = y + jnp.dot(W0d_r[...], dn.astype(jnp.bfloat16),
                        preferred_element_type=jnp.float32)
        y = y + b0_r[...]
        y0_r[...] = y
        se = jnp.dot(S_r[...], x, preferred_element_type=jnp.float32)
        sq = jnp.dot(S_r[...], x * x, preferred_element_type=jnp.float32)
        fm = 0.5 * jnp.sum(se * se - sq, axis=0, keepdims=True)
        fo = jnp.sum(fo_r[...], axis=0, keepdims=True)
        fo = fo + jnp.dot(Wd_r[...], dn, preferred_element_type=jnp.float32)
        fofm_r[...] = fm + fo + cb_r[...]
        s1 = jnp.sum(y, axis=1, keepdims=True)
        s2 = jnp.sum(y * y, axis=1, keepdims=True)

        @pl.when(i == 0)
        def _():
            ss_r[...] = s1
            sq_r[...] = s2

        @pl.when(i != 0)
        def _():
            ss_r[...] = ss_r[...] + s1
            sq_r[...] = sq_r[...] + s2

    return pl.pallas_call(
        body,
        grid=(nb,),
        in_specs=[
            pl.BlockSpec((E, bb), lambda i: (0, i)),
            pl.BlockSpec((fo_t.shape[0], bb), lambda i: (0, i)),
            pl.BlockSpec((dense_t.shape[0], bb), lambda i: (0, i)),
            pl.BlockSpec(S_t.shape, lambda i: (0, 0)),
            pl.BlockSpec(W0e.shape, lambda i: (0, 0)),
            pl.BlockSpec(W0d.shape, lambda i: (0, 0)),
            pl.BlockSpec(b0c.shape, lambda i: (0, 0)),
            pl.BlockSpec(Wdr.shape, lambda i: (0, 0)),
            pl.BlockSpec(cbias.shape, lambda i: (0, 0)),
        ],
        out_specs=[
            pl.BlockSpec((H, bb), lambda i: (0, i)),
            pl.BlockSpec((H, 1), lambda i: (0, 0)),
            pl.BlockSpec((H, 1), lambda i: (0, 0)),
            pl.BlockSpec((1, bb), lambda i: (0, i)),
        ],
        out_shape=[
            jax.ShapeDtypeStruct((H, Bn), jnp.float32),
            jax.ShapeDtypeStruct((H, 1), jnp.float32),
            jax.ShapeDtypeStruct((H, 1), jnp.float32),
            jax.ShapeDtypeStruct((1, Bn), jnp.float32),
        ],
    )(emb_t, fo_t, dense_t, S_t, W0e, W0d, b0c, Wdr, cbias)


def _stage_hidden_t(yp, ss, sq, gc, bec, W, bc, bb):
    H, Bn = yp.shape
    H1 = W.shape[0]
    nb = Bn // bb
    inv_b = 1.0 / Bn

    def body(yp_r, ss_r, sq_r, g_r, be_r, W_r, b_r, y_r, oss_r, osq_r):
        i = pl.program_id(0)
        mean = ss_r[...] * inv_b
        var = sq_r[...] * inv_b - mean * mean
        scale = g_r[...] * lax.rsqrt(var + _EPS)
        shift = be_r[...] - mean * scale
        h = jnp.maximum(yp_r[...] * scale + shift, 0.0)
        y = jnp.dot(W_r[...], h.astype(jnp.bfloat16),
                    preferred_element_type=jnp.float32) + b_r[...]
        y_r[...] = y
        s1 = jnp.sum(y, axis=1, keepdims=True)
        s2 = jnp.sum(y * y, axis=1, keepdims=True)

        @pl.when(i == 0)
        def _():
            oss_r[...] = s1
            osq_r[...] = s2

        @pl.when(i != 0)
        def _():
            oss_r[...] = oss_r[...] + s1
            osq_r[...] = osq_r[...] + s2

    return pl.pallas_call(
        body,
        grid=(nb,),
        in_specs=[
            pl.BlockSpec((H, bb), lambda i: (0, i)),
            pl.BlockSpec((H, 1), lambda i: (0, 0)),
            pl.BlockSpec((H, 1), lambda i: (0, 0)),
            pl.BlockSpec(gc.shape, lambda i: (0, 0)),
            pl.BlockSpec(bec.shape, lambda i: (0, 0)),
            pl.BlockSpec(W.shape, lambda i: (0, 0)),
            pl.BlockSpec(bc.shape, lambda i: (0, 0)),
        ],
        out_specs=[
            pl.BlockSpec((H1, bb), lambda i: (0, i)),
            pl.BlockSpec((H1, 1), lambda i: (0, 0)),
            pl.BlockSpec((H1, 1), lambda i: (0, 0)),
        ],
        out_shape=[
            jax.ShapeDtypeStruct((H1, Bn), jnp.float32),
            jax.ShapeDtypeStruct((H1, 1), jnp.float32),
            jax.ShapeDtypeStruct((H1, 1), jnp.float32),
        ],
    )(yp, ss, sq, gc, bec, W, bc)


def _stage_final_t(yp, ss, sq, gc, bec, Wor, fofm, cbias, bb):
    H, Bn = yp.shape
    nb = Bn // bb
    inv_b = 1.0 / Bn

    def body(yp_r, ss_r, sq_r, g_r, be_r, Wo_r, ff_r, cb_r, out_r):
        mean = ss_r[...] * inv_b
        var = sq_r[...] * inv_b - mean * mean
        scale = g_r[...] * lax.rsqrt(var + _EPS)
        shift = be_r[...] - mean * scale
        h = jnp.maximum(yp_r[...] * scale + shift, 0.0)
        z = jnp.dot(Wo_r[...], h, preferred_element_type=jnp.float32)
        z = z + ff_r[...] + cb_r[...]
        out_r[...] = 1.0 / (1.0 + jnp.exp(-z))

    return pl.pallas_call(
        body,
        grid=(nb,),
        in_specs=[
            pl.BlockSpec((H, bb), lambda i: (0, i)),
            pl.BlockSpec((H, 1), lambda i: (0, 0)),
            pl.BlockSpec((H, 1), lambda i: (0, 0)),
            pl.BlockSpec(gc.shape, lambda i: (0, 0)),
            pl.BlockSpec(bec.shape, lambda i: (0, 0)),
            pl.BlockSpec(Wor.shape, lambda i: (0, 0)),
            pl.BlockSpec((1, bb), lambda i: (0, i)),
            pl.BlockSpec(cbias.shape, lambda i: (0, 0)),
        ],
        out_specs=pl.BlockSpec((1, bb), lambda i: (0, i)),
        out_shape=jax.ShapeDtypeStruct((1, Bn), jnp.float32),
    )(yp, ss, sq, gc, bec, Wor, fofm, cbias)


def kernel(sparse_inputs, dense_inputs, emb1, emb2, Wd, bd,
           W0, b0, g0, beta0, W1, b1, g1, beta1, W2, b2, g2, beta2,
           Wo, bo, bias0):
    Bn, F = sparse_inputs.shape
    _, V, D = emb2.shape
    E = F * D

    emb2_rows = emb2.transpose(0, 2, 1).reshape(E, V)
    emb1_rows = emb1.transpose(0, 2, 1).reshape(F, V)
    idx_t = sparse_inputs.T
    dense_t = dense_inputs.T

    emb_t, fo_t = _sc_gather_t(emb2_rows, emb1_rows, idx_t)

    S_t = jnp.tile(jnp.eye(D, dtype=jnp.float32), (1, F))
    W0e = W0[:, :E].astype(jnp.bfloat16)
    W0d = W0[:, E:].astype(jnp.bfloat16)
    W1b = W1.astype(jnp.bfloat16)
    W2b = W2.astype(jnp.bfloat16)
    cbias_a = (bd + bias0).reshape(1, 1)

    bb = 2048
    y0, ss0, sq0, fofm = _stage_first_t(emb_t, fo_t, dense_t, S_t,
                                        W0e, W0d, b0.reshape(-1, 1), Wd,
                                        cbias_a, bb)
    y1, ss1, sq1 = _stage_hidden_t(y0, ss0, sq0, g0.reshape(-1, 1),
                                   beta0.reshape(-1, 1), W1b,
                                   b1.reshape(-1, 1), bb)
    y2, ss2, sq2 = _stage_hidden_t(y1, ss1, sq1, g1.reshape(-1, 1),
                                   beta1.reshape(-1, 1), W2b,
                                   b2.reshape(-1, 1), bb)
    out = _stage_final_t(y2, ss2, sq2, g2.reshape(-1, 1),
                         beta2.reshape(-1, 1), Wo, fofm,
                         bo.reshape(1, 1), bb)
    return out.reshape(Bn)


# speedup vs baseline: 5.2918x; 1.0624x over previous
import functools

import jax
import jax.numpy as jnp
from jax import lax
from jax.experimental import pallas as pl
from jax.experimental.pallas import tpu as pltpu
from jax.experimental.pallas import tpu_sc as plsc

_EPS = 1e-5
_NC = 2
_NS = 16
_NW = _NC * _NS


def _sc_gather_t(emb2_rows, emb1_rows, idx_t):
    R2, V = emb2_rows.shape
    F, B = idx_t.shape
    D = R2 // F
    NJ = R2 // _NW
    QC = 4096
    NQ = B // QC
    GRP = 8

    @functools.partial(
        pl.kernel,
        mesh=plsc.VectorSubcoreMesh(core_axis_name="c", subcore_axis_name="s"),
        compiler_params=pltpu.CompilerParams(use_tc_tiling_on_sc=True,
                                             needs_layout_passes=False),
        out_type=[
            jax.ShapeDtypeStruct((R2, B), jnp.float32),
            jax.ShapeDtypeStruct((F, B), jnp.float32),
        ],
        scratch_types=[
            pltpu.VMEM((V,), jnp.float32),
            pltpu.VMEM((B,), jnp.int32),
            pltpu.VMEM((QC,), jnp.float32),
            pltpu.VMEM((QC,), jnp.float32),
            pltpu.SemaphoreType.DMA,
            pltpu.SemaphoreType.DMA,
            pltpu.SemaphoreType.DMA,
        ],
    )
    def k(e2_h, e1_h, idx_h, out2_h, out1_h,
          row_v, idx_v, ob0, ob1, sem_in, semw0, semw1):
        wid = lax.axis_index("s") * _NC + lax.axis_index("c")
        r0 = wid * NJ
        obufs = (ob0, ob1)
        semws = (semw0, semw1)
        pending = [None, None]

        def gather_quarter(q, ob):
            def body(i, _):
                base = q * QC + i * (GRP * 16)
                obase = i * (GRP * 16)
                for g in range(GRP):
                    iv = idx_v[pl.ds(base + g * 16, 16)]
                    ob[pl.ds(obase + g * 16, 16)] = plsc.load_gather(row_v, [iv])
                return 0
            lax.fori_loop(0, QC // (GRP * 16), body, 0)

        def do_row(out_h, r):
            for q in range(NQ):
                kq = q % 2
                if pending[kq] is not None:
                    pending[kq].wait()
                gather_quarter(q, obufs[kq])
                pending[kq] = pltpu.async_copy(
                    obufs[kq], out_h.at[r, pl.ds(q * QC, QC)], semws[kq])

        for j in range(NJ):
            r = r0 + j
            f = r // D
            if j == 0:
                pltpu.sync_copy(idx_h.at[f], idx_v)
            else:
                fprev = (r0 + j - 1) // D

                @pl.when(f != fprev)
                def _():
                    pltpu.sync_copy(idx_h.at[f], idx_v)

            pltpu.sync_copy(e2_h.at[r], row_v)
            do_row(out2_h, r)

        for kq in range(2):
            if pending[kq] is not None:
                pending[kq].wait()

        @pl.when(wid < F)
        def _():
            pltpu.sync_copy(idx_h.at[wid], idx_v)
            pltpu.sync_copy(e1_h.at[wid], row_v)
            for q in range(NQ):
                gather_quarter(q, obufs[q % 2])
                pltpu.sync_copy(obufs[q % 2],
                                out1_h.at[wid, pl.ds(q * QC, QC)])

    return k(emb2_rows, emb1_rows, idx_t)


def _stage_first_t(emb_t, fo_t, dense_t, S_t, W0e, W0d, b0c, Wdr, cbias, bb):
    E, Bn = emb_t.shape
    H = W0e.shape[0]
    nb = Bn // bb

    def body(emb_r, fo_r, dn_r, S_r, W0e_r, W0d_r, b0_r, Wd_r, cb_r,
             y0_r, ss_r, sq_r, fofm_r):
        i = pl.program_id(0)
        x = emb_r[...]
        dn = dn_r[...]
        xb = x.astype(jnp.bfloat16)
        y = jnp.dot(W0e_r[...], xb, preferred_element_type=jnp.float32)
        y = y + jnp.dot(W0d_r[...], dn.astype(jnp.bfloat16),
                        preferred_element_type=jnp.float32)
        y = y + b0_r[...]
        y0_r[...] = y.astype(jnp.bfloat16)
        se = jnp.dot(S_r[...], x, preferred_element_type=jnp.float32)
        sq = jnp.dot(S_r[...], x * x, preferred_element_type=jnp.float32)
        fm = 0.5 * jnp.sum(se * se - sq, axis=0, keepdims=True)
        fo = jnp.sum(fo_r[...], axis=0, keepdims=True)
        fo = fo + jnp.dot(Wd_r[...], dn, preferred_element_type=jnp.float32)
        fofm_r[...] = fm + fo + cb_r[...]
        s1 = jnp.sum(y, axis=1, keepdims=True)
        s2 = jnp.sum(y * y, axis=1, keepdims=True)

        @pl.when(i == 0)
        def _():
            ss_r[...] = s1
            sq_r[...] = s2

        @pl.when(i != 0)
        def _():
            ss_r[...] = ss_r[...] + s1
            sq_r[...] = sq_r[...] + s2

    return pl.pallas_call(
        body,
        grid=(nb,),
        in_specs=[
            pl.BlockSpec((E, bb), lambda i: (0, i)),
            pl.BlockSpec((fo_t.shape[0], bb), lambda i: (0, i)),
            pl.BlockSpec((dense_t.shape[0], bb), lambda i: (0, i)),
            pl.BlockSpec(S_t.shape, lambda i: (0, 0)),
            pl.BlockSpec(W0e.shape, lambda i: (0, 0)),
            pl.BlockSpec(W0d.shape, lambda i: (0, 0)),
            pl.BlockSpec(b0c.shape, lambda i: (0, 0)),
            pl.BlockSpec(Wdr.shape, lambda i: (0, 0)),
            pl.BlockSpec(cbias.shape, lambda i: (0, 0)),
        ],
        out_specs=[
            pl.BlockSpec((H, bb), lambda i: (0, i)),
            pl.BlockSpec((H, 1), lambda i: (0, 0)),
            pl.BlockSpec((H, 1), lambda i: (0, 0)),
            pl.BlockSpec((1, bb), lambda i: (0, i)),
        ],
        out_shape=[
            jax.ShapeDtypeStruct((H, Bn), jnp.bfloat16),
            jax.ShapeDtypeStruct((H, 1), jnp.float32),
            jax.ShapeDtypeStruct((H, 1), jnp.float32),
            jax.ShapeDtypeStruct((1, Bn), jnp.float32),
        ],
    )(emb_t, fo_t, dense_t, S_t, W0e, W0d, b0c, Wdr, cbias)


def _stage_hidden_t(yp, ss, sq, gc, bec, W, bc, bb):
    H, Bn = yp.shape
    H1 = W.shape[0]
    nb = Bn // bb
    inv_b = 1.0 / Bn

    def body(yp_r, ss_r, sq_r, g_r, be_r, W_r, b_r, y_r, oss_r, osq_r):
        i = pl.program_id(0)
        mean = ss_r[...] * inv_b
        var = sq_r[...] * inv_b - mean * mean
        scale = g_r[...] * lax.rsqrt(var + _EPS)
        shift = be_r[...] - mean * scale
        h = jnp.maximum(yp_r[...].astype(jnp.float32) * scale + shift, 0.0)
        y = jnp.dot(W_r[...], h.astype(jnp.bfloat16),
                    preferred_element_type=jnp.float32) + b_r[...]
        y_r[...] = y.astype(jnp.bfloat16)
        s1 = jnp.sum(y, axis=1, keepdims=True)
        s2 = jnp.sum(y * y, axis=1, keepdims=True)

        @pl.when(i == 0)
        def _():
            oss_r[...] = s1
            osq_r[...] = s2

        @pl.when(i != 0)
        def _():
            oss_r[...] = oss_r[...] + s1
            osq_r[...] = osq_r[...] + s2

    return pl.pallas_call(
        body,
        grid=(nb,),
        in_specs=[
            pl.BlockSpec((H, bb), lambda i: (0, i)),
            pl.BlockSpec((H, 1), lambda i: (0, 0)),
            pl.BlockSpec((H, 1), lambda i: (0, 0)),
            pl.BlockSpec(gc.shape, lambda i: (0, 0)),
            pl.BlockSpec(bec.shape, lambda i: (0, 0)),
            pl.BlockSpec(W.shape, lambda i: (0, 0)),
            pl.BlockSpec(bc.shape, lambda i: (0, 0)),
        ],
        out_specs=[
            pl.BlockSpec((H1, bb), lambda i: (0, i)),
            pl.BlockSpec((H1, 1), lambda i: (0, 0)),
            pl.BlockSpec((H1, 1), lambda i: (0, 0)),
        ],
        out_shape=[
            jax.ShapeDtypeStruct((H1, Bn), jnp.bfloat16),
            jax.ShapeDtypeStruct((H1, 1), jnp.float32),
            jax.ShapeDtypeStruct((H1, 1), jnp.float32),
        ],
    )(yp, ss, sq, gc, bec, W, bc)


def _stage_final_t(yp, ss, sq, gc, bec, Wor, fofm, cbias, bb):
    H, Bn = yp.shape
    nb = Bn // bb
    inv_b = 1.0 / Bn

    def body(yp_r, ss_r, sq_r, g_r, be_r, Wo_r, ff_r, cb_r, out_r):
        mean = ss_r[...] * inv_b
        var = sq_r[...] * inv_b - mean * mean
        scale = g_r[...] * lax.rsqrt(var + _EPS)
        shift = be_r[...] - mean * scale
        h = jnp.maximum(yp_r[...].astype(jnp.float32) * scale + shift, 0.0)
        z = jnp.dot(Wo_r[...], h, preferred_element_type=jnp.float32)
        z = z + ff_r[...] + cb_r[...]
        out_r[...] = 1.0 / (1.0 + jnp.exp(-z))

    return pl.pallas_call(
        body,
        grid=(nb,),
        in_specs=[
            pl.BlockSpec((H, bb), lambda i: (0, i)),
            pl.BlockSpec((H, 1), lambda i: (0, 0)),
            pl.BlockSpec((H, 1), lambda i: (0, 0)),
            pl.BlockSpec(gc.shape, lambda i: (0, 0)),
            pl.BlockSpec(bec.shape, lambda i: (0, 0)),
            pl.BlockSpec(Wor.shape, lambda i: (0, 0)),
            pl.BlockSpec((1, bb), lambda i: (0, i)),
            pl.BlockSpec(cbias.shape, lambda i: (0, 0)),
        ],
        out_specs=pl.BlockSpec((1, bb), lambda i: (0, i)),
        out_shape=jax.ShapeDtypeStruct((1, Bn), jnp.float32),
    )(yp, ss, sq, gc, bec, Wor, fofm, cbias)


def kernel(sparse_inputs, dense_inputs, emb1, emb2, Wd, bd,
           W0, b0, g0, beta0, W1, b1, g1, beta1, W2, b2, g2, beta2,
           Wo, bo, bias0):
    Bn, F = sparse_inputs.shape
    _, V, D = emb2.shape
    E = F * D

    emb2_rows = emb2.transpose(0, 2, 1).reshape(E, V)
    emb1_rows = emb1.transpose(0, 2, 1).reshape(F, V)
    idx_t = sparse_inputs.T
    dense_t = dense_inputs.T

    emb_t, fo_t = _sc_gather_t(emb2_rows, emb1_rows, idx_t)

    S_t = jnp.tile(jnp.eye(D, dtype=jnp.float32), (1, F))
    W0e = W0[:, :E].astype(jnp.bfloat16)
    W0d = W0[:, E:].astype(jnp.bfloat16)
    W1b = W1.astype(jnp.bfloat16)
    W2b = W2.astype(jnp.bfloat16)
    cbias_a = (bd + bias0).reshape(1, 1)

    bb = 2048
    y0, ss0, sq0, fofm = _stage_first_t(emb_t, fo_t, dense_t, S_t,
                                        W0e, W0d, b0.reshape(-1, 1), Wd,
                                        cbias_a, bb)
    y1, ss1, sq1 = _stage_hidden_t(y0, ss0, sq0, g0.reshape(-1, 1),
                                   beta0.reshape(-1, 1), W1b,
                                   b1.reshape(-1, 1), bb)
    y2, ss2, sq2 = _stage_hidden_t(y1, ss1, sq1, g1.reshape(-1, 1),
                                   beta1.reshape(-1, 1), W2b,
                                   b2.reshape(-1, 1), bb)
    out = _stage_final_t(y2, ss2, sq2, g2.reshape(-1, 1),
                         beta2.reshape(-1, 1), Wo, fofm,
                         bo.reshape(1, 1), bb)
    return out.reshape(Bn)


# speedup vs baseline: 5.3674x; 1.0143x over previous
import functools

import jax
import jax.numpy as jnp
from jax import lax
from jax.experimental import pallas as pl
from jax.experimental.pallas import tpu as pltpu
from jax.experimental.pallas import tpu_sc as plsc

_EPS = 1e-5
_NC = 2
_NS = 16
_NW = _NC * _NS


def _sc_gather_t(emb2_rows, emb1_rows, idx_t):
    R2, V = emb2_rows.shape
    F, B = idx_t.shape
    D = R2 // F
    NJ = R2 // _NW
    QC = 4096
    NQ = B // QC
    GRP = 8

    @functools.partial(
        pl.kernel,
        mesh=plsc.VectorSubcoreMesh(core_axis_name="c", subcore_axis_name="s"),
        compiler_params=pltpu.CompilerParams(use_tc_tiling_on_sc=True,
                                             needs_layout_passes=False),
        out_type=[
            jax.ShapeDtypeStruct((R2, B), jnp.float32),
            jax.ShapeDtypeStruct((F, B), jnp.float32),
        ],
        scratch_types=[
            pltpu.VMEM((V,), jnp.float32),
            pltpu.VMEM((B,), jnp.int32),
            pltpu.VMEM((QC,), jnp.float32),
            pltpu.VMEM((QC,), jnp.float32),
            pltpu.SemaphoreType.DMA,
            pltpu.SemaphoreType.DMA,
            pltpu.SemaphoreType.DMA,
        ],
    )
    def k(e2_h, e1_h, idx_h, out2_h, out1_h,
          row_v, idx_v, ob0, ob1, sem_in, semw0, semw1):
        wid = lax.axis_index("s") * _NC + lax.axis_index("c")
        r0 = wid * NJ
        obufs = (ob0, ob1)
        semws = (semw0, semw1)
        pending = [None, None]

        def gather_quarter(q, ob):
            def body(i, _):
                base = q * QC + i * (GRP * 16)
                obase = i * (GRP * 16)
                for g in range(GRP):
                    iv = idx_v[pl.ds(base + g * 16, 16)]
                    ob[pl.ds(obase + g * 16, 16)] = plsc.load_gather(row_v, [iv])
                return 0
            lax.fori_loop(0, QC // (GRP * 16), body, 0)

        def do_row(out_h, r):
            for q in range(NQ):
                kq = q % 2
                if pending[kq] is not None:
                    pending[kq].wait()
                gather_quarter(q, obufs[kq])
                pending[kq] = pltpu.async_copy(
                    obufs[kq], out_h.at[r, pl.ds(q * QC, QC)], semws[kq])

        for j in range(NJ):
            r = r0 + j
            f = r // D
            if j == 0:
                pltpu.sync_copy(idx_h.at[f], idx_v)
            else:
                fprev = (r0 + j - 1) // D

                @pl.when(f != fprev)
                def _():
                    pltpu.sync_copy(idx_h.at[f], idx_v)

            pltpu.sync_copy(e2_h.at[r], row_v)
            do_row(out2_h, r)

        for kq in range(2):
            if pending[kq] is not None:
                pending[kq].wait()

        @pl.when(wid < F)
        def _():
            pltpu.sync_copy(idx_h.at[wid], idx_v)
            pltpu.sync_copy(e1_h.at[wid], row_v)
            for q in range(NQ):
                gather_quarter(q, obufs[q % 2])
                pltpu.sync_copy(obufs[q % 2],
                                out1_h.at[wid, pl.ds(q * QC, QC)])

    return k(emb2_rows, emb1_rows, idx_t)


def _mlp_fused_t(emb_t, fo_t, dense_t, S_t, W0e, W0d, b0c, Wdr, cba,
                 g0c, be0c, W1b, b1c, g1c, be1c, W2b, b2c, g2c, be2c,
                 Wor, cbo, bb):
    E, Bn = emb_t.shape
    H = W0e.shape[0]
    nb = Bn // bb
    inv_b = 1.0 / Bn
    f32 = jnp.float32
    bf16 = jnp.bfloat16

    def body(emb_r, fo_r, dn_r, S_r, W0e_r, W0d_r, b0_r, Wd_r, cba_r,
             g0_r, be0_r, W1_r, b1_r, g1_r, be1_r, W2_r, b2_r, g2_r, be2_r,
             Wo_r, cbo_r, out_r, y0s, y1s, y2s, sts, fof):
        s = pl.program_id(0)
        i = pl.program_id(1)
        sl = pl.ds(i * bb, bb)

        def acc_stats(col, y):
            s1 = jnp.sum(y, axis=1, keepdims=True)
            s2 = jnp.sum(y * y, axis=1, keepdims=True)

            @pl.when(i == 0)
            def _():
                sts[:, col:col + 1] = s1
                sts[:, col + 1:col + 2] = s2

            @pl.when(i != 0)
            def _():
                sts[:, col:col + 1] = sts[:, col:col + 1] + s1
                sts[:, col + 1:col + 2] = sts[:, col + 1:col + 2] + s2

        def bn_relu(ys, col, g_r, be_r):
            mean = sts[:, col:col + 1] * inv_b
            var = sts[:, col + 1:col + 2] * inv_b - mean * mean
            scale = g_r[...] * lax.rsqrt(var + _EPS)
            shift = be_r[...] - mean * scale
            yp = ys[:, sl].astype(f32)
            return jnp.maximum(yp * scale + shift, 0.0)

        @pl.when(s == 0)
        def _():
            x = emb_r[...]
            dn = dn_r[...]
            y = jnp.dot(W0e_r[...], x.astype(bf16), preferred_element_type=f32)
            y = y + jnp.dot(W0d_r[...], dn.astype(bf16),
                            preferred_element_type=f32)
            y = y + b0_r[...]
            y0s[:, sl] = y.astype(bf16)
            acc_stats(0, y)
            se = jnp.dot(S_r[...], x, preferred_element_type=f32)
            sq = jnp.dot(S_r[...], x * x, preferred_element_type=f32)
            fm = 0.5 * jnp.sum(se * se - sq, axis=0, keepdims=True)
            fo = jnp.sum(fo_r[...], axis=0, keepdims=True)
            fo = fo + jnp.dot(Wd_r[...], dn, preferred_element_type=f32)
            fof[:, sl] = fm + fo + cba_r[...]

        @pl.when(s == 1)
        def _():
            h = bn_relu(y0s, 0, g0_r, be0_r)
            y = jnp.dot(W1_r[...], h.astype(bf16),
                        preferred_element_type=f32) + b1_r[...]
            y1s[:, sl] = y.astype(bf16)
            acc_stats(2, y)

        @pl.when(s == 2)
        def _():
            h = bn_relu(y1s, 2, g1_r, be1_r)
            y = jnp.dot(W2_r[...], h.astype(bf16),
                        preferred_element_type=f32) + b2_r[...]
            y2s[:, sl] = y.astype(bf16)
            acc_stats(4, y)

        @pl.when(s == 3)
        def _():
            h = bn_relu(y2s, 4, g2_r, be2_r)
            z = jnp.dot(Wo_r[...], h, preferred_element_type=f32)
            z = z + fof[:, sl] + cbo_r[...]
            out_r[...] = 1.0 / (1.0 + jnp.exp(-z))

    const = lambda shape: pl.BlockSpec(shape, lambda s, i: (0, 0))
    blk = lambda rows: pl.BlockSpec(
        (rows, bb), lambda s, i: (0, jnp.where(s == 0, i, nb - 1)))
    return pl.pallas_call(
        body,
        grid=(4, nb),
        in_specs=[
            blk(E),
            blk(fo_t.shape[0]),
            blk(dense_t.shape[0]),
            const(S_t.shape),
            const(W0e.shape),
            const(W0d.shape),
            const(b0c.shape),
            const(Wdr.shape),
            const(cba.shape),
            const(g0c.shape),
            const(be0c.shape),
            const(W1b.shape),
            const(b1c.shape),
            const(g1c.shape),
            const(be1c.shape),
            const(W2b.shape),
            const(b2c.shape),
            const(g2c.shape),
            const(be2c.shape),
            const(Wor.shape),
            const(cbo.shape),
        ],
        out_specs=pl.BlockSpec((1, bb),
                               lambda s, i: (0, jnp.where(s == 3, i, 0))),
        out_shape=jax.ShapeDtypeStruct((1, Bn), jnp.float32),
        scratch_shapes=[
            pltpu.VMEM((H, Bn), jnp.bfloat16),
            pltpu.VMEM((H, Bn), jnp.bfloat16),
            pltpu.VMEM((H, Bn), jnp.bfloat16),
            pltpu.VMEM((H, 8), jnp.float32),
            pltpu.VMEM((1, Bn), jnp.float32),
        ],
    )(emb_t, fo_t, dense_t, S_t, W0e, W0d, b0c, Wdr, cba,
      g0c, be0c, W1b, b1c, g1c, be1c, W2b, b2c, g2c, be2c, Wor, cbo)


def kernel(sparse_inputs, dense_inputs, emb1, emb2, Wd, bd,
           W0, b0, g0, beta0, W1, b1, g1, beta1, W2, b2, g2, beta2,
           Wo, bo, bias0):
    Bn, F = sparse_inputs.shape
    _, V, D = emb2.shape
    E = F * D

    emb2_rows = emb2.transpose(0, 2, 1).reshape(E, V)
    emb1_rows = emb1.transpose(0, 2, 1).reshape(F, V)
    idx_t = sparse_inputs.T
    dense_t = dense_inputs.T

    emb_t, fo_t = _sc_gather_t(emb2_rows, emb1_rows, idx_t)

    S_t = jnp.tile(jnp.eye(D, dtype=jnp.float32), (1, F))
    W0e = W0[:, :E].astype(jnp.bfloat16)
    W0d = W0[:, E:].astype(jnp.bfloat16)
    W1b = W1.astype(jnp.bfloat16)
    W2b = W2.astype(jnp.bfloat16)
    cbias_a = (bd + bias0).reshape(1, 1)

    bb = 2048
    out = _mlp_fused_t(emb_t, fo_t, dense_t, S_t, W0e, W0d,
                       b0.reshape(-1, 1), Wd, cbias_a,
                       g0.reshape(-1, 1), beta0.reshape(-1, 1),
                       W1b, b1.reshape(-1, 1),
                       g1.reshape(-1, 1), beta1.reshape(-1, 1),
                       W2b, b2.reshape(-1, 1),
                       g2.reshape(-1, 1), beta2.reshape(-1, 1),
                       Wo, bo.reshape(1, 1), bb)
    return out.reshape(Bn)
